# baseline (speedup 1.0000x reference)
import jax
import jax.numpy as jnp
from jax.experimental import pallas as pl
from jax.experimental.pallas import tpu as pltpu

_EPS = 1e-12
_TN = (((0,), (0,)), ((), ()))
_TIA = 400
_TI = 1000


def _pass_a(x_ref, w1_ref, b1_ref, h_ref, g1t_ref, de_ref, h16_ref, dvs_ref):
    i = pl.program_id(0)
    panel = h_ref[...]
    p16 = panel.astype(jnp.bfloat16)
    h16_ref[...] = p16
    x1 = jnp.dot(x_ref[...], w1_ref[...],
                 preferred_element_type=jnp.float32) + b1_ref[...]
    dv = jnp.sum(panel, axis=1, keepdims=True)
    dvs = jax.lax.rsqrt(dv + _EPS)
    dvs_ref[...] = jnp.broadcast_to(dvs, dvs_ref.shape)
    a1 = (x1 * dvs).astype(jnp.bfloat16)
    g1t_blk = jax.lax.dot_general(a1, p16, _TN,
                                  preferred_element_type=jnp.float32)
    de_blk = jnp.sum(panel, axis=0, keepdims=True)

    @pl.when(i == 0)
    def _init():
        g1t_ref[...] = g1t_blk
        de_ref[...] = de_blk

    @pl.when(i > 0)
    def _acc():
        g1t_ref[...] += g1t_blk
        de_ref[...] += de_blk


def _pass_b(h_ref, dvs_ref, b1m_ref, w2_ref, b2_ref, g2t_ref):
    i = pl.program_id(0)
    panel = h_ref[...]
    dvs = dvs_ref[:, :1]
    y = jnp.dot(panel, b1m_ref[...],
                preferred_element_type=jnp.float32)
    y = jnp.maximum(y * dvs, 0.0)
    x2 = jnp.dot(y, w2_ref[...],
                 preferred_element_type=jnp.float32) + b2_ref[...]
    a2 = (x2 * dvs).astype(jnp.bfloat16)
    g2t_blk = jax.lax.dot_general(a2, panel, _TN,
                                  preferred_element_type=jnp.float32)

    @pl.when(i == 0)
    def _init():
        g2t_ref[...] = g2t_blk

    @pl.when(i > 0)
    def _acc():
        g2t_ref[...] += g2t_blk


def _pass_c(h_ref, dvs_ref, b2m_ref, out_ref):
    panel = h_ref[...]
    dvs = dvs_ref[:, :1]
    out_ref[...] = dvs * jnp.dot(panel, b2m_ref[...],
                                 preferred_element_type=jnp.float32)


def kernel(X, H, W1, b1, W2, b2):
    n, c_in = X.shape
    m = H.shape[1]
    c_hid = W1.shape[1]
    c_out = W2.shape[1]
    assert n % _TIA == 0 and n % _TI == 0
    nba = n // _TIA
    nb = n // _TI
    b1r = b1.reshape(1, c_hid)
    b2r = b2.reshape(1, c_out)

    params = pltpu.CompilerParams(dimension_semantics=("arbitrary",))

    g1t, de, h16, dvsb = pl.pallas_call(
        _pass_a,
        grid=(nba,),
        in_specs=[
            pl.BlockSpec((_TIA, c_in), lambda i: (i, 0)),
            pl.BlockSpec((c_in, c_hid), lambda i: (0, 0)),
            pl.BlockSpec((1, c_hid), lambda i: (0, 0)),
            pl.BlockSpec((_TIA, m), lambda i: (i, 0)),
        ],
        out_specs=[
            pl.BlockSpec((c_hid, m), lambda i: (0, 0)),
            pl.BlockSpec((1, m), lambda i: (0, 0)),
            pl.BlockSpec((_TIA, m), lambda i: (i, 0)),
            pl.BlockSpec((_TIA, 128), lambda i: (i, 0)),
        ],
        out_shape=[
            jax.ShapeDtypeStruct((c_hid, m), jnp.float32),
            jax.ShapeDtypeStruct((1, m), jnp.float32),
            jax.ShapeDtypeStruct((n, m), jnp.bfloat16),
            jax.ShapeDtypeStruct((n, 128), jnp.float32),
        ],
        compiler_params=params,
    )(X, W1, b1r, H)

    de_inv_col = (1.0 / (de[0] + _EPS))[:, None]
    b1m = (g1t.T * de_inv_col).astype(jnp.bfloat16)

    g2t = pl.pallas_call(
        _pass_b,
        grid=(nb,),
        in_specs=[
            pl.BlockSpec((_TI, m), lambda i: (i, 0)),
            pl.BlockSpec((_TI, 128), lambda i: (i, 0)),
            pl.BlockSpec((m, c_hid), lambda i: (0, 0)),
            pl.BlockSpec((c_hid, c_out), lambda i: (0, 0)),
            pl.BlockSpec((1, c_out), lambda i: (0, 0)),
        ],
        out_specs=pl.BlockSpec((c_out, m), lambda i: (0, 0)),
        out_shape=jax.ShapeDtypeStruct((c_out, m), jnp.float32),
        compiler_params=params,
    )(h16, dvsb, b1m, W2, b2r)

    b2m = (g2t.T * de_inv_col).astype(jnp.bfloat16)

    x2 = pl.pallas_call(
        _pass_c,
        grid=(nb,),
        in_specs=[
            pl.BlockSpec((_TI, m), lambda i: (i, 0)),
            pl.BlockSpec((_TI, 128), lambda i: (i, 0)),
            pl.BlockSpec((m, c_out), lambda i: (0, 0)),
        ],
        out_specs=pl.BlockSpec((_TI, c_out), lambda i: (i, 0)),
        out_shape=jax.ShapeDtypeStruct((n, c_out), jnp.float32),
        compiler_params=params,
    )(h16, dvsb, b2m)

    return x2


# speedup vs baseline: 1.2334x; 1.2334x over previous
import jax
import jax.numpy as jnp
from jax.experimental import pallas as pl
from jax.experimental.pallas import tpu as pltpu

_EPS = 1e-12
_TN = (((0,), (0,)), ((), ()))
_TI = 1000


def _pass_a(x_ref, w1_ref, b1_ref, h_ref, g1t_ref, de_ref, dvs_ref):
    i = pl.program_id(0)
    panel = h_ref[...]
    x1 = jnp.dot(x_ref[...], w1_ref[...],
                 preferred_element_type=jnp.float32) + b1_ref[...]
    dv = jnp.sum(panel, axis=1, keepdims=True,
                 dtype=jnp.float32)
    dvs = jax.lax.rsqrt(dv + _EPS)
    dvs_ref[...] = jnp.broadcast_to(dvs, dvs_ref.shape)
    a1 = (x1 * dvs).astype(jnp.bfloat16)
    g1t_blk = jax.lax.dot_general(a1, panel, _TN,
                                  preferred_element_type=jnp.float32)
    de_blk = jnp.sum(panel, axis=0, keepdims=True,
                     dtype=jnp.float32)

    @pl.when(i == 0)
    def _init():
        g1t_ref[...] = g1t_blk
        de_ref[...] = de_blk

    @pl.when(i > 0)
    def _acc():
        g1t_ref[...] += g1t_blk
        de_ref[...] += de_blk


def _pass_b(h_ref, dvs_ref, b1m_ref, w2_ref, b2_ref, g2t_ref):
    i = pl.program_id(0)
    panel = h_ref[...]
    dvs = dvs_ref[:, :1]
    y = jnp.dot(panel, b1m_ref[...],
                preferred_element_type=jnp.float32)
    y = jnp.maximum(y * dvs, 0.0)
    x2 = jnp.dot(y, w2_ref[...],
                 preferred_element_type=jnp.float32) + b2_ref[...]
    a2 = (x2 * dvs).astype(jnp.bfloat16)
    g2t_blk = jax.lax.dot_general(a2, panel, _TN,
                                  preferred_element_type=jnp.float32)

    @pl.when(i == 0)
    def _init():
        g2t_ref[...] = g2t_blk

    @pl.when(i > 0)
    def _acc():
        g2t_ref[...] += g2t_blk


def _pass_c(h_ref, dvs_ref, b2m_ref, out_ref):
    panel = h_ref[...]
    dvs = dvs_ref[:, :1]
    out_ref[...] = dvs * jnp.dot(panel, b2m_ref[...],
                                 preferred_element_type=jnp.float32)


def kernel(X, H, W1, b1, W2, b2):
    n, c_in = X.shape
    m = H.shape[1]
    c_hid = W1.shape[1]
    c_out = W2.shape[1]
    ti = _TI
    assert n % ti == 0
    nb = n // ti
    b1r = b1.reshape(1, c_hid)
    b2r = b2.reshape(1, c_out)
    h16 = H.astype(jnp.bfloat16)

    params = pltpu.CompilerParams(dimension_semantics=("arbitrary",))

    g1t, de, dvsb = pl.pallas_call(
        _pass_a,
        grid=(nb,),
        in_specs=[
            pl.BlockSpec((ti, c_in), lambda i: (i, 0)),
            pl.BlockSpec((c_in, c_hid), lambda i: (0, 0)),
            pl.BlockSpec((1, c_hid), lambda i: (0, 0)),
            pl.BlockSpec((ti, m), lambda i: (i, 0)),
        ],
        out_specs=[
            pl.BlockSpec((c_hid, m), lambda i: (0, 0)),
            pl.BlockSpec((1, m), lambda i: (0, 0)),
            pl.BlockSpec((ti, 128), lambda i: (i, 0)),
        ],
        out_shape=[
            jax.ShapeDtypeStruct((c_hid, m), jnp.float32),
            jax.ShapeDtypeStruct((1, m), jnp.float32),
            jax.ShapeDtypeStruct((n, 128), jnp.float32),
        ],
        compiler_params=params,
    )(X, W1, b1r, h16)

    de_inv_col = (1.0 / (de[0] + _EPS))[:, None]
    b1m = (g1t.T * de_inv_col).astype(jnp.bfloat16)

    g2t = pl.pallas_call(
        _pass_b,
        grid=(nb,),
        in_specs=[
            pl.BlockSpec((ti, m), lambda i: (i, 0)),
            pl.BlockSpec((ti, 128), lambda i: (i, 0)),
            pl.BlockSpec((m, c_hid), lambda i: (0, 0)),
            pl.BlockSpec((c_hid, c_out), lambda i: (0, 0)),
            pl.BlockSpec((1, c_out), lambda i: (0, 0)),
        ],
        out_specs=pl.BlockSpec((c_out, m), lambda i: (0, 0)),
        out_shape=jax.ShapeDtypeStruct((c_out, m), jnp.float32),
        compiler_params=params,
    )(h16, dvsb, b1m, W2, b2r)

    b2m = (g2t.T * de_inv_col).astype(jnp.bfloat16)

    x2 = pl.pallas_call(
        _pass_c,
        grid=(nb,),
        in_specs=[
            pl.BlockSpec((ti, m), lambda i: (i, 0)),
            pl.BlockSpec((ti, 128), lambda i: (i, 0)),
            pl.BlockSpec((m, c_out), lambda i: (0, 0)),
        ],
        out_specs=pl.BlockSpec((ti, c_out), lambda i: (i, 0)),
        out_shape=jax.ShapeDtypeStruct((n, c_out), jnp.float32),
        compiler_params=params,
    )(h16, dvsb, b2m)

    return x2
